# Initial kernel scaffold; baseline (speedup 1.0000x reference)
import jax
import jax.numpy as jnp
from jax.experimental import pallas as pl


def kernel(features, adj_indices, adj_values, weight, bias, skip_weight):
    raise NotImplementedError("write your pallas kernel here")


# speedup vs baseline: 3.9125x; 3.9125x over previous
import functools

import jax
import jax.numpy as jnp
from jax import lax
from jax.experimental import pallas as pl
from jax.experimental.pallas import tpu as pltpu
from jax.experimental.pallas import tpu_sc as plsc

NC = 2
NS = 16
LANES = 16
CHUNK = 128

_SELU_ALPHA = 1.6732632423543772
_SELU_SCALE = 1.0507009873554805


def _matmul_body(x_ref, w_ref, o_ref):
    o_ref[...] = jnp.dot(x_ref[...], w_ref[...],
                         preferred_element_type=jnp.float32)


def _finalize_body(h_ref, p_ref, skip_ref, bias_ref, o_ref):
    x = (h_ref[...] * skip_ref[...] + p_ref[0] + p_ref[1] + bias_ref[...])
    o_ref[...] = _SELU_SCALE * jnp.where(
        x > 0.0, x, _SELU_ALPHA * (jnp.exp(x) - 1.0))


def _sc_aggregate(h, col, row, val, zeros_rows, n_rows, d, e_pad):
    ept = e_pad // (NC * NS)
    n_chunks = ept // CHUNK
    rows_per_tile = n_rows // NS

    mesh = plsc.VectorSubcoreMesh(core_axis_name="c", subcore_axis_name="s")

    @functools.partial(
        pl.kernel,
        out_type=jax.ShapeDtypeStruct((NC, n_rows, d), jnp.float32),
        mesh=mesh,
        scratch_types=[
            pltpu.VMEM_SHARED((n_rows, d), jnp.float32),
            pltpu.VMEM((CHUNK,), jnp.int32),
            pltpu.VMEM((CHUNK,), jnp.int32),
            pltpu.VMEM((CHUNK,), jnp.float32),
            pltpu.VMEM((CHUNK, d), jnp.float32),
            pltpu.SemaphoreType.DMA,
        ],
    )
    def agg(h_hbm, col_hbm, row_hbm, val_hbm, zeros_hbm, out_hbm,
            acc, colv, rowv, valv, rowsv, sem):
        c = lax.axis_index("c")
        s = lax.axis_index("s")
        wid = c * NS + s

        r0 = s * rows_per_tile
        pltpu.sync_copy(zeros_hbm.at[pl.ds(0, rows_per_tile)],
                        acc.at[pl.ds(r0, rows_per_tile)])
        plsc.subcore_barrier()

        base = wid * ept

        def chunk_body(i, carry):
            off = base + i * CHUNK
            pltpu.sync_copy(col_hbm.at[pl.ds(off, CHUNK)], colv)
            pltpu.sync_copy(row_hbm.at[pl.ds(off, CHUNK)], rowv)
            pltpu.sync_copy(val_hbm.at[pl.ds(off, CHUNK)], valv)
            pltpu.async_copy(h_hbm.at[colv], rowsv, sem).wait()

            def group_body(g, carry2):
                vv = valv[pl.ds(g * LANES, LANES)]
                for e in range(LANES):
                    r = g * LANES + e
                    v = vv[e]
                    for k in range(d // LANES):
                        sl = pl.ds(k * LANES, LANES)
                        rowsv[r, sl] = rowsv[r, sl] * v
                return carry2

            lax.fori_loop(0, CHUNK // LANES, group_body, 0)
            pltpu.sync_copy(rowsv, acc.at[rowv], add=True)
            return carry

        lax.fori_loop(0, n_chunks, chunk_body, 0)

        plsc.subcore_barrier()
        pltpu.sync_copy(acc.at[pl.ds(r0, rows_per_tile)],
                        out_hbm.at[c, pl.ds(r0, rows_per_tile)])

    return agg(h, col, row, val, zeros_rows)


def kernel(features, adj_indices, adj_values, weight, bias, skip_weight):
    n, d_in = features.shape
    d = weight.shape[1]
    e = adj_values.shape[0]

    h = pl.pallas_call(
        _matmul_body,
        out_shape=jax.ShapeDtypeStruct((n, d), jnp.float32),
    )(features, weight)

    group = NC * NS * CHUNK
    e_pad = ((e + group - 1) // group) * group
    pad = e_pad - e
    row = adj_indices[0].astype(jnp.int32)
    col = adj_indices[1].astype(jnp.int32)
    val = adj_values
    if pad:
        zi = jnp.zeros((pad,), jnp.int32)
        row = jnp.concatenate([row, zi])
        col = jnp.concatenate([col, zi])
        val = jnp.concatenate([val, jnp.zeros((pad,), jnp.float32)])

    n_acc = ((n + NS * 8 - 1) // (NS * 8)) * (NS * 8)
    zeros_rows = jnp.zeros((n_acc // NS, d), jnp.float32)

    partials = _sc_aggregate(h, col, row, val, zeros_rows, n_acc, d, e_pad)
    partials = partials[:, :n, :]

    blk = 2000
    out = pl.pallas_call(
        _finalize_body,
        grid=(n // blk,),
        in_specs=[
            pl.BlockSpec((blk, d), lambda i: (i, 0)),
            pl.BlockSpec((NC, blk, d), lambda i: (0, i, 0)),
            pl.BlockSpec((1, d), lambda i: (0, 0)),
            pl.BlockSpec((1, d), lambda i: (0, 0)),
        ],
        out_specs=pl.BlockSpec((blk, d), lambda i: (i, 0)),
        out_shape=jax.ShapeDtypeStruct((n, d), jnp.float32),
    )(h, partials, skip_weight.reshape(1, d), bias.reshape(1, d))
    return out


# speedup vs baseline: 4.3162x; 1.1032x over previous
import functools

import jax
import jax.numpy as jnp
from jax import lax
from jax.experimental import pallas as pl
from jax.experimental.pallas import tpu as pltpu
from jax.experimental.pallas import tpu_sc as plsc

NC = 2
NS = 16
LANES = 16
CHUNK = 64
NBUF = 4

_SELU_ALPHA = 1.6732632423543772
_SELU_SCALE = 1.0507009873554805


def _matmul_body(x_ref, w_ref, o_ref):
    o_ref[...] = jnp.dot(x_ref[...], w_ref[...],
                         preferred_element_type=jnp.float32)


def _finalize_body(h_ref, p_ref, skip_ref, bias_ref, o_ref):
    x = (h_ref[...] * skip_ref[...] + p_ref[0] + p_ref[1] + bias_ref[...])
    o_ref[...] = _SELU_SCALE * jnp.where(
        x > 0.0, x, _SELU_ALPHA * (jnp.exp(x) - 1.0))


def _sc_aggregate(h, col, row, val, zeros_rows, n_rows, d, e_pad):
    ept = e_pad // (NC * NS)
    nch = ept // CHUNK
    assert nch % NBUF == 0
    rows_per_tile = n_rows // NS

    mesh = plsc.VectorSubcoreMesh(core_axis_name="c", subcore_axis_name="s")

    @functools.partial(
        pl.kernel,
        out_type=jax.ShapeDtypeStruct((NC, n_rows, d), jnp.float32),
        mesh=mesh,
        scratch_types=[
            pltpu.VMEM_SHARED((n_rows, d), jnp.float32),
            [pltpu.VMEM((CHUNK,), jnp.int32) for _ in range(NBUF)],
            [pltpu.VMEM((CHUNK,), jnp.int32) for _ in range(NBUF)],
            [pltpu.VMEM((CHUNK,), jnp.float32) for _ in range(NBUF)],
            [pltpu.VMEM((CHUNK, d), jnp.float32) for _ in range(NBUF)],
            [pltpu.SemaphoreType.DMA for _ in range(NBUF)],
            [pltpu.SemaphoreType.DMA for _ in range(NBUF)],
            [pltpu.SemaphoreType.DMA for _ in range(NBUF)],
            [pltpu.SemaphoreType.DMA for _ in range(NBUF)],
            [pltpu.SemaphoreType.DMA for _ in range(NBUF)],
        ],
    )
    def agg(h_hbm, col_hbm, row_hbm, val_hbm, zeros_hbm, out_hbm,
            acc, colv, rowv, valv, bufs, csem, rsem, vsem, gsem, ssem):
        c = lax.axis_index("c")
        s = lax.axis_index("s")
        wid = c * NS + s

        r0 = s * rows_per_tile
        pltpu.sync_copy(zeros_hbm.at[pl.ds(0, rows_per_tile)],
                        acc.at[pl.ds(r0, rows_per_tile)])
        plsc.subcore_barrier()

        base = wid * ept

        def col_copy(ch, b):
            return pltpu.make_async_copy(
                col_hbm.at[pl.ds(base + ch * CHUNK, CHUNK)], colv[b], csem[b])

        def row_copy(ch, b):
            return pltpu.make_async_copy(
                row_hbm.at[pl.ds(base + ch * CHUNK, CHUNK)], rowv[b], rsem[b])

        def val_copy(ch, b):
            return pltpu.make_async_copy(
                val_hbm.at[pl.ds(base + ch * CHUNK, CHUNK)], valv[b], vsem[b])

        def gather(b):
            return pltpu.make_async_copy(h_hbm.at[colv[b]], bufs[b], gsem[b])

        def idx_issue(ch, b):
            col_copy(ch, b).start()
            val_copy(ch, b).start()

        def scatter_start(b):
            pltpu.async_copy(bufs[b], acc.at[rowv[b]], ssem[b], add=True)

        def scatter_wait(b):
            pltpu.make_async_copy(bufs[b], acc.at[rowv[b]], ssem[b]).wait()

        idx_issue(0, 0)
        idx_issue(1, 1)
        idx_issue(2, 2)
        row_copy(0, 0).start()
        row_copy(1, 1).start()
        col_copy(0, 0).wait()
        gather(0).start()
        col_copy(1, 1).wait()
        gather(1).start()

        def quad_body(i, carry):
            for b in range(NBUF):
                ch = i * NBUF + b
                b2 = (b + 2) % NBUF
                b3 = (b + 3) % NBUF

                @pl.when(ch >= 2)
                def _():
                    scatter_wait(b2)

                @pl.when(ch + 2 < nch)
                def _():
                    row_copy(ch + 2, b2).start()
                    col_copy(ch + 2, b2).wait()
                    gather(b2).start()

                @pl.when(ch + 3 < nch)
                def _():
                    idx_issue(ch + 3, b3)

                gather(b).wait()
                val_copy(ch, b).wait()

                def group_body(g, carry2):
                    vv = valv[b][pl.ds(g * LANES, LANES)]
                    for e in range(LANES):
                        v = vv[e]
                        for k in range(d // LANES):
                            sl = pl.ds(k * LANES, LANES)
                            bufs[b][g * LANES + e, sl] = (
                                bufs[b][g * LANES + e, sl] * v)
                    return carry2

                lax.fori_loop(0, CHUNK // LANES, group_body, 0)
                row_copy(ch, b).wait()
                scatter_start(b)
            return carry

        lax.fori_loop(0, nch // NBUF, quad_body, 0)

        scatter_wait((nch - 2) % NBUF)
        scatter_wait((nch - 1) % NBUF)

        plsc.subcore_barrier()
        pltpu.sync_copy(acc.at[pl.ds(r0, rows_per_tile)],
                        out_hbm.at[c, pl.ds(r0, rows_per_tile)])

    return agg(h, col, row, val, zeros_rows)


def kernel(features, adj_indices, adj_values, weight, bias, skip_weight):
    n, d_in = features.shape
    d = weight.shape[1]
    e = adj_values.shape[0]

    h = pl.pallas_call(
        _matmul_body,
        out_shape=jax.ShapeDtypeStruct((n, d), jnp.float32),
    )(features, weight)

    group = NC * NS * CHUNK * NBUF
    e_pad = ((e + group - 1) // group) * group
    pad = e_pad - e
    row = adj_indices[0].astype(jnp.int32)
    col = adj_indices[1].astype(jnp.int32)
    val = adj_values
    if pad:
        zi = jnp.zeros((pad,), jnp.int32)
        row = jnp.concatenate([row, zi])
        col = jnp.concatenate([col, zi])
        val = jnp.concatenate([val, jnp.zeros((pad,), jnp.float32)])

    n_acc = ((n + NS * 8 - 1) // (NS * 8)) * (NS * 8)
    zeros_rows = jnp.zeros((n_acc // NS, d), jnp.float32)

    partials = _sc_aggregate(h, col, row, val, zeros_rows, n_acc, d, e_pad)
    partials = partials[:, :n, :]

    blk = 2000
    out = pl.pallas_call(
        _finalize_body,
        grid=(n // blk,),
        in_specs=[
            pl.BlockSpec((blk, d), lambda i: (i, 0)),
            pl.BlockSpec((NC, blk, d), lambda i: (0, i, 0)),
            pl.BlockSpec((1, d), lambda i: (0, 0)),
            pl.BlockSpec((1, d), lambda i: (0, 0)),
        ],
        out_specs=pl.BlockSpec((blk, d), lambda i: (i, 0)),
        out_shape=jax.ShapeDtypeStruct((n, d), jnp.float32),
    )(h, partials, skip_weight.reshape(1, d), bias.reshape(1, d))
    return out


# speedup vs baseline: 8.7844x; 2.0352x over previous
import functools

import jax
import jax.numpy as jnp
from jax import lax
from jax.experimental import pallas as pl
from jax.experimental.pallas import tpu as pltpu
from jax.experimental.pallas import tpu_sc as plsc

NC = 2
NS = 16
LANES = 16
CHUNK = 112
NBUF = 4
OCH = 64
SCH = 32

_SELU_ALPHA = 1.6732632423543772
_SELU_SCALE = 1.0507009873554805


def _matmul_body(x_ref, w_ref, o_ref):
    o_ref[...] = jnp.dot(x_ref[...], w_ref[...],
                         preferred_element_type=jnp.float32)


def _finalize_body(h_ref, p_ref, skip_ref, bias_ref, o_ref):
    ah = jnp.concatenate([p_ref[0], p_ref[1]], axis=-1)
    x = h_ref[...] * skip_ref[...] + ah + bias_ref[...]
    o_ref[...] = _SELU_SCALE * jnp.where(
        x > 0.0, x, _SELU_ALPHA * (jnp.exp(x) - 1.0))


def _sc_aggregate(h, col, row, val, n_rows, d, e_pad):
    dh = d // NC
    ept = e_pad // NS
    nch = ept // CHUNK
    assert nch % NBUF == 0
    rows_per_tile = n_rows // NS
    assert rows_per_tile % OCH == 0 and rows_per_tile % SCH == 0

    mesh = plsc.VectorSubcoreMesh(core_axis_name="c", subcore_axis_name="s")

    @functools.partial(
        pl.kernel,
        out_type=jax.ShapeDtypeStruct((NC, n_rows // 2, d), jnp.float32),
        mesh=mesh,
        compiler_params=pltpu.CompilerParams(use_tc_tiling_on_sc=False),
        scratch_types=[
            pltpu.VMEM_SHARED((n_rows, dh), jnp.float32),
            pltpu.VMEM_SHARED((n_rows, dh), jnp.float32),
            [pltpu.VMEM((CHUNK,), jnp.int32) for _ in range(NBUF)],
            [pltpu.VMEM((CHUNK,), jnp.int32) for _ in range(NBUF)],
            [pltpu.VMEM((CHUNK,), jnp.float32) for _ in range(NBUF)],
            [pltpu.VMEM((CHUNK, dh), jnp.float32) for _ in range(NBUF)],
            pltpu.VMEM((SCH, d), jnp.float32),
            pltpu.VMEM((OCH,), jnp.int32),
            pltpu.SemaphoreType.DMA,
            [pltpu.SemaphoreType.DMA for _ in range(NBUF)],
            [pltpu.SemaphoreType.DMA for _ in range(NBUF)],
            [pltpu.SemaphoreType.DMA for _ in range(NBUF)],
            [pltpu.SemaphoreType.DMA for _ in range(NBUF)],
            [pltpu.SemaphoreType.DMA for _ in range(NBUF)],
        ],
    )
    def agg(h_hbm, col_hbm, row_hbm, val_hbm, out_hbm,
            h_sp, acc, colv, rowv, valv, bufs, bounce, idxz,
            osem, csem, rsem, vsem, gsem, ssem):
        c = lax.axis_index("c")
        s = lax.axis_index("s")
        r0 = s * rows_per_tile
        coff = c * dh
        iota = lax.iota(jnp.int32, LANES)

        def fill_idxz(nrows, base):
            for g in range(nrows // LANES):
                idxz[pl.ds(g * LANES, LANES)] = iota + (base + g * LANES)

        for z in range(rows_per_tile // OCH):
            for half in range(OCH // SCH):
                pltpu.sync_copy(
                    h_hbm.at[pl.ds(r0 + z * OCH + half * SCH, SCH)], bounce)

                def stage_body(i, carry):
                    for k in range(dh // LANES):
                        bufs[0][half * SCH + i, pl.ds(k * LANES, LANES)] = (
                            bounce[i, pl.ds(coff + k * LANES, LANES)])
                    return carry

                lax.fori_loop(0, SCH, stage_body, 0)
            fill_idxz(OCH, r0 + z * OCH)
            pltpu.sync_copy(bufs[0].at[pl.ds(0, OCH)], h_sp.at[idxz])

        def zero_body(i, carry):
            for k in range(dh // LANES):
                bufs[0][i, pl.ds(k * LANES, LANES)] = jnp.zeros(
                    (LANES,), jnp.float32)
            return carry

        lax.fori_loop(0, OCH, zero_body, 0)
        for z in range(rows_per_tile // OCH):
            fill_idxz(OCH, r0 + z * OCH)
            pltpu.sync_copy(bufs[0].at[pl.ds(0, OCH)], acc.at[idxz])
        plsc.subcore_barrier()

        base = s * ept

        def col_copy(ch, b):
            return pltpu.make_async_copy(
                col_hbm.at[pl.ds(base + ch * CHUNK, CHUNK)], colv[b], csem[b])

        def row_copy(ch, b):
            return pltpu.make_async_copy(
                row_hbm.at[pl.ds(base + ch * CHUNK, CHUNK)], rowv[b], rsem[b])

        def val_copy(ch, b):
            return pltpu.make_async_copy(
                val_hbm.at[pl.ds(base + ch * CHUNK, CHUNK)], valv[b], vsem[b])

        def gather(b):
            return pltpu.make_async_copy(h_sp.at[colv[b]], bufs[b], gsem[b])

        def idx_issue(ch, b):
            col_copy(ch, b).start()
            val_copy(ch, b).start()

        def scatter_start(b):
            pltpu.async_copy(bufs[b], acc.at[rowv[b]], ssem[b], add=True)

        def scatter_wait(b):
            pltpu.make_async_copy(bufs[b], acc.at[rowv[b]], ssem[b]).wait()

        idx_issue(0, 0)
        idx_issue(1, 1)
        idx_issue(2, 2)
        row_copy(0, 0).start()
        row_copy(1, 1).start()
        col_copy(0, 0).wait()
        gather(0).start()
        col_copy(1, 1).wait()
        gather(1).start()

        def quad_body(i, carry):
            for b in range(NBUF):
                ch = i * NBUF + b
                b2 = (b + 2) % NBUF
                b3 = (b + 3) % NBUF

                @pl.when(ch >= 2)
                def _():
                    scatter_wait(b2)

                @pl.when(ch + 2 < nch)
                def _():
                    row_copy(ch + 2, b2).start()
                    col_copy(ch + 2, b2).wait()
                    gather(b2).start()

                @pl.when(ch + 3 < nch)
                def _():
                    idx_issue(ch + 3, b3)

                gather(b).wait()
                val_copy(ch, b).wait()

                def group_body(g, carry2):
                    vv = valv[b][pl.ds(g * LANES, LANES)]
                    for e in range(LANES):
                        r = g * LANES + e
                        v = vv[e]
                        for k in range(dh // LANES):
                            sl = pl.ds(k * LANES, LANES)
                            bufs[b][r, sl] = bufs[b][r, sl] * v
                    return carry2

                lax.fori_loop(0, CHUNK // LANES, group_body, 0)
                row_copy(ch, b).wait()
                scatter_start(b)
            return carry

        lax.fori_loop(0, nch // NBUF, quad_body, 0)

        scatter_wait((nch - 2) % NBUF)
        scatter_wait((nch - 1) % NBUF)

        plsc.subcore_barrier()

        for z in range(rows_per_tile // OCH):
            fill_idxz(OCH, r0 + z * OCH)
            pltpu.sync_copy(acc.at[idxz], bufs[0].at[pl.ds(0, OCH)])
            for q in range(OCH // (2 * SCH)):

                def pack_body(i, carry):
                    for m in range(d // LANES):
                        bounce[i, pl.ds(m * LANES, LANES)] = bufs[0][
                            2 * SCH * q + 2 * i + m // (dh // LANES),
                            pl.ds((m % (dh // LANES)) * LANES, LANES)]
                    return carry

                lax.fori_loop(0, SCH, pack_body, 0)
                o0 = (s * (rows_per_tile // 2) + z * (OCH // 2) + q * SCH)
                pltpu.sync_copy(bounce, out_hbm.at[c, pl.ds(o0, SCH)])

    return agg(h, col, row, val)


def kernel(features, adj_indices, adj_values, weight, bias, skip_weight):
    n, d_in = features.shape
    d = weight.shape[1]
    e = adj_values.shape[0]

    n_acc = ((n + NS * OCH - 1) // (NS * OCH)) * (NS * OCH)
    feat = jnp.concatenate(
        [features, jnp.zeros((n_acc - n, d_in), jnp.float32)])

    h = pl.pallas_call(
        _matmul_body,
        out_shape=jax.ShapeDtypeStruct((n_acc, d), jnp.float32),
    )(feat, weight)

    group = NS * CHUNK * NBUF
    e_pad = ((e + group - 1) // group) * group
    pad = e_pad - e
    row = adj_indices[0].astype(jnp.int32)
    col = adj_indices[1].astype(jnp.int32)
    val = adj_values
    if pad:
        spread = n + jnp.arange(pad, dtype=jnp.int32) % (n_acc - n)
        row = jnp.concatenate([row, spread])
        col = jnp.concatenate([col, spread])
        val = jnp.concatenate([val, jnp.zeros((pad,), jnp.float32)])

    partials = _sc_aggregate(h, col, row, val, n_acc, d, e_pad)
    partials = partials.reshape(NC, n_acc, d // NC)[:, :n, :]

    blk = 2000
    out = pl.pallas_call(
        _finalize_body,
        grid=(n // blk,),
        in_specs=[
            pl.BlockSpec((blk, d), lambda i: (i, 0)),
            pl.BlockSpec((NC, blk, d // NC), lambda i: (0, i, 0)),
            pl.BlockSpec((1, d), lambda i: (0, 0)),
            pl.BlockSpec((1, d), lambda i: (0, 0)),
        ],
        out_specs=pl.BlockSpec((blk, d), lambda i: (i, 0)),
        out_shape=jax.ShapeDtypeStruct((n, d), jnp.float32),
    )(h[:n], partials, skip_weight.reshape(1, d), bias.reshape(1, d))
    return out


# speedup vs baseline: 9.9644x; 1.1343x over previous
import functools

import jax
import jax.numpy as jnp
from jax import lax
from jax.experimental import pallas as pl
from jax.experimental.pallas import tpu as pltpu
from jax.experimental.pallas import tpu_sc as plsc

NC = 2
NS = 16
LANES = 16
CHUNK = 112
NBUF = 4
OCH = 64
SCH = 32

_SELU_ALPHA = 1.6732632423543772
_SELU_SCALE = 1.0507009873554805


def _matmul_body(x_ref, w_ref, o_ref):
    o_ref[...] = jnp.dot(x_ref[...], w_ref[...],
                         preferred_element_type=jnp.float32)


def _finalize_body(h_ref, p_ref, skip_ref, bias_ref, o_ref):
    ah = jnp.concatenate([p_ref[0], p_ref[1]], axis=-1)
    x = h_ref[...] * skip_ref[...] + ah + bias_ref[...]
    o_ref[...] = _SELU_SCALE * jnp.where(
        x > 0.0, x, _SELU_ALPHA * (jnp.exp(x) - 1.0))


def _sc_aggregate(h, col, row, val, n_rows, d, e_pad):
    dh = d // NC
    ept = e_pad // NS
    nch = ept // CHUNK
    assert nch % NBUF == 0
    rows_per_tile = n_rows // NS
    assert rows_per_tile % OCH == 0 and rows_per_tile % SCH == 0

    mesh = plsc.VectorSubcoreMesh(core_axis_name="c", subcore_axis_name="s")

    @functools.partial(
        pl.kernel,
        out_type=jax.ShapeDtypeStruct((NC, n_rows // 2, d), jnp.float32),
        mesh=mesh,
        compiler_params=pltpu.CompilerParams(use_tc_tiling_on_sc=False),
        scratch_types=[
            pltpu.VMEM_SHARED((n_rows, dh), jnp.float32),
            pltpu.VMEM_SHARED((n_rows, dh), jnp.float32),
            [pltpu.VMEM((CHUNK,), jnp.int32) for _ in range(NBUF)],
            [pltpu.VMEM((CHUNK,), jnp.int32) for _ in range(NBUF)],
            [pltpu.VMEM((CHUNK,), jnp.float32) for _ in range(NBUF)],
            [pltpu.VMEM((CHUNK, dh), jnp.float32) for _ in range(NBUF)],
            pltpu.VMEM((SCH, d), jnp.float32),
            pltpu.VMEM((OCH,), jnp.int32),
            pltpu.SemaphoreType.DMA,
            [pltpu.SemaphoreType.DMA for _ in range(NBUF)],
            [pltpu.SemaphoreType.DMA for _ in range(NBUF)],
            [pltpu.SemaphoreType.DMA for _ in range(NBUF)],
            [pltpu.SemaphoreType.DMA for _ in range(NBUF)],
            [pltpu.SemaphoreType.DMA for _ in range(NBUF)],
        ],
    )
    def agg(h_hbm, col_hbm, row_hbm, val_hbm, out_hbm,
            h_sp, acc, colv, rowv, valv, bufs, bounce, idxz,
            osem, csem, rsem, vsem, gsem, ssem):
        c = lax.axis_index("c")
        s = lax.axis_index("s")
        r0 = s * rows_per_tile
        coff = c * dh
        iota = lax.iota(jnp.int32, LANES)

        def fill_idxz(nrows, base):
            for g in range(nrows // LANES):
                idxz[pl.ds(g * LANES, LANES)] = iota + (base + g * LANES)

        pltpu.async_copy(
            h_hbm.at[pl.ds(r0, rows_per_tile), pl.ds(coff, dh)],
            h_sp.at[pl.ds(r0, rows_per_tile)], osem)

        def zero_body(i, carry):
            for k in range(dh // LANES):
                bufs[0][i, pl.ds(k * LANES, LANES)] = jnp.zeros(
                    (LANES,), jnp.float32)
            return carry

        lax.fori_loop(0, OCH, zero_body, 0)
        for z in range(rows_per_tile // OCH):
            fill_idxz(OCH, r0 + z * OCH)
            pltpu.sync_copy(bufs[0].at[pl.ds(0, OCH)], acc.at[idxz])
        pltpu.make_async_copy(
            h_hbm.at[pl.ds(r0, rows_per_tile), pl.ds(coff, dh)],
            h_sp.at[pl.ds(r0, rows_per_tile)], osem).wait()
        plsc.subcore_barrier()

        base = s * ept

        def col_copy(ch, b):
            return pltpu.make_async_copy(
                col_hbm.at[pl.ds(base + ch * CHUNK, CHUNK)], colv[b], csem[b])

        def row_copy(ch, b):
            return pltpu.make_async_copy(
                row_hbm.at[pl.ds(base + ch * CHUNK, CHUNK)], rowv[b], rsem[b])

        def val_copy(ch, b):
            return pltpu.make_async_copy(
                val_hbm.at[pl.ds(base + ch * CHUNK, CHUNK)], valv[b], vsem[b])

        def gather(b):
            return pltpu.make_async_copy(h_sp.at[colv[b]], bufs[b], gsem[b])

        def idx_issue(ch, b):
            col_copy(ch, b).start()
            val_copy(ch, b).start()

        def scatter_start(b):
            pltpu.async_copy(bufs[b], acc.at[rowv[b]], ssem[b], add=True)

        def scatter_wait(b):
            pltpu.make_async_copy(bufs[b], acc.at[rowv[b]], ssem[b]).wait()

        idx_issue(0, 0)
        idx_issue(1, 1)
        idx_issue(2, 2)
        row_copy(0, 0).start()
        row_copy(1, 1).start()
        col_copy(0, 0).wait()
        gather(0).start()
        col_copy(1, 1).wait()
        gather(1).start()

        def quad_body(i, carry):
            for b in range(NBUF):
                ch = i * NBUF + b
                b2 = (b + 2) % NBUF
                b3 = (b + 3) % NBUF

                @pl.when(ch >= 2)
                def _():
                    scatter_wait(b2)

                @pl.when(ch + 2 < nch)
                def _():
                    row_copy(ch + 2, b2).start()
                    col_copy(ch + 2, b2).wait()
                    gather(b2).start()

                @pl.when(ch + 3 < nch)
                def _():
                    idx_issue(ch + 3, b3)

                gather(b).wait()
                val_copy(ch, b).wait()

                def group_body(g, carry2):
                    vv = valv[b][pl.ds(g * LANES, LANES)]
                    for e in range(LANES):
                        r = g * LANES + e
                        v = vv[e]
                        for k in range(dh // LANES):
                            sl = pl.ds(k * LANES, LANES)
                            bufs[b][r, sl] = bufs[b][r, sl] * v
                    return carry2

                lax.fori_loop(0, CHUNK // LANES, group_body, 0)
                row_copy(ch, b).wait()
                scatter_start(b)
            return carry

        lax.fori_loop(0, nch // NBUF, quad_body, 0)

        scatter_wait((nch - 2) % NBUF)
        scatter_wait((nch - 1) % NBUF)

        plsc.subcore_barrier()

        for z in range(rows_per_tile // OCH):
            fill_idxz(OCH, r0 + z * OCH)
            pltpu.sync_copy(acc.at[idxz], bufs[0].at[pl.ds(0, OCH)])
            for q in range(OCH // (2 * SCH)):

                def pack_body(i, carry):
                    for m in range(d // LANES):
                        bounce[i, pl.ds(m * LANES, LANES)] = bufs[0][
                            2 * SCH * q + 2 * i + m // (dh // LANES),
                            pl.ds((m % (dh // LANES)) * LANES, LANES)]
                    return carry

                lax.fori_loop(0, SCH, pack_body, 0)
                o0 = (s * (rows_per_tile // 2) + z * (OCH // 2) + q * SCH)
                pltpu.sync_copy(bounce, out_hbm.at[c, pl.ds(o0, SCH)])

    return agg(h, col, row, val)


def kernel(features, adj_indices, adj_values, weight, bias, skip_weight):
    n, d_in = features.shape
    d = weight.shape[1]
    e = adj_values.shape[0]

    n_acc = ((n + NS * OCH - 1) // (NS * OCH)) * (NS * OCH)
    feat = jnp.concatenate(
        [features, jnp.zeros((n_acc - n, d_in), jnp.float32)])

    h = pl.pallas_call(
        _matmul_body,
        out_shape=jax.ShapeDtypeStruct((n_acc, d), jnp.float32),
    )(feat, weight)

    group = NS * CHUNK * NBUF
    e_pad = ((e + group - 1) // group) * group
    pad = e_pad - e
    row = adj_indices[0].astype(jnp.int32)
    col = adj_indices[1].astype(jnp.int32)
    val = adj_values
    if pad:
        spread = n + jnp.arange(pad, dtype=jnp.int32) % (n_acc - n)
        row = jnp.concatenate([row, spread])
        col = jnp.concatenate([col, spread])
        val = jnp.concatenate([val, jnp.zeros((pad,), jnp.float32)])

    partials = _sc_aggregate(h, col, row, val, n_acc, d, e_pad)
    partials = partials.reshape(NC, n_acc, d // NC)[:, :n, :]

    blk = 2000
    out = pl.pallas_call(
        _finalize_body,
        grid=(n // blk,),
        in_specs=[
            pl.BlockSpec((blk, d), lambda i: (i, 0)),
            pl.BlockSpec((NC, blk, d // NC), lambda i: (0, i, 0)),
            pl.BlockSpec((1, d), lambda i: (0, 0)),
            pl.BlockSpec((1, d), lambda i: (0, 0)),
        ],
        out_specs=pl.BlockSpec((blk, d), lambda i: (i, 0)),
        out_shape=jax.ShapeDtypeStruct((n, d), jnp.float32),
    )(h[:n], partials, skip_weight.reshape(1, d), bias.reshape(1, d))
    return out


# speedup vs baseline: 10.1093x; 1.0145x over previous
import functools

import jax
import jax.numpy as jnp
from jax import lax
from jax.experimental import pallas as pl
from jax.experimental.pallas import tpu as pltpu
from jax.experimental.pallas import tpu_sc as plsc

NC = 2
NS = 16
LANES = 16
CHUNK = 112
NBUF = 4
OCH = 64
SCH = 32

_SELU_ALPHA = 1.6732632423543772
_SELU_SCALE = 1.0507009873554805


def _matmul_body(x_ref, w_ref, o_ref):
    o_ref[...] = jnp.dot(x_ref[...], w_ref[...],
                         preferred_element_type=jnp.float32)


def _finalize_body(h_ref, p_ref, skip_ref, bias_ref, o_ref):
    ah = jnp.concatenate([p_ref[0], p_ref[1]], axis=-1)
    x = h_ref[...] * skip_ref[...] + ah + bias_ref[...]
    o_ref[...] = _SELU_SCALE * jnp.where(
        x > 0.0, x, _SELU_ALPHA * (jnp.exp(x) - 1.0))


def _sc_aggregate(h, col, row, val, n_rows, d, e_pad):
    dh = d // NC
    ept = e_pad // NS
    nch = ept // CHUNK
    assert nch % NBUF == 0
    rows_per_tile = n_rows // NS
    assert rows_per_tile % OCH == 0 and rows_per_tile % SCH == 0

    mesh = plsc.VectorSubcoreMesh(core_axis_name="c", subcore_axis_name="s")

    @functools.partial(
        pl.kernel,
        out_type=jax.ShapeDtypeStruct((NC, n_rows, dh), jnp.float32),
        mesh=mesh,
        compiler_params=pltpu.CompilerParams(use_tc_tiling_on_sc=False),
        scratch_types=[
            pltpu.VMEM_SHARED((n_rows, dh), jnp.float32),
            pltpu.VMEM_SHARED((n_rows, dh), jnp.float32),
            [pltpu.VMEM((CHUNK,), jnp.int32) for _ in range(NBUF)],
            [pltpu.VMEM((CHUNK,), jnp.int32) for _ in range(NBUF)],
            [pltpu.VMEM((CHUNK,), jnp.float32) for _ in range(NBUF)],
            [pltpu.VMEM((CHUNK, dh), jnp.float32) for _ in range(NBUF)],
            pltpu.VMEM((SCH, d), jnp.float32),
            pltpu.VMEM((OCH,), jnp.int32),
            pltpu.SemaphoreType.DMA,
            [pltpu.SemaphoreType.DMA for _ in range(NBUF)],
            [pltpu.SemaphoreType.DMA for _ in range(NBUF)],
            [pltpu.SemaphoreType.DMA for _ in range(NBUF)],
            [pltpu.SemaphoreType.DMA for _ in range(NBUF)],
            [pltpu.SemaphoreType.DMA for _ in range(NBUF)],
        ],
    )
    def agg(h_hbm, col_hbm, row_hbm, val_hbm, out_hbm,
            h_sp, acc, colv, rowv, valv, bufs, bounce, idxz,
            osem, csem, rsem, vsem, gsem, ssem):
        c = lax.axis_index("c")
        s = lax.axis_index("s")
        r0 = s * rows_per_tile
        coff = c * dh
        iota = lax.iota(jnp.int32, LANES)

        def fill_idxz(nrows, base):
            for g in range(nrows // LANES):
                idxz[pl.ds(g * LANES, LANES)] = iota + (base + g * LANES)

        pltpu.async_copy(
            h_hbm.at[pl.ds(r0, rows_per_tile), pl.ds(coff, dh)],
            h_sp.at[pl.ds(r0, rows_per_tile)], osem)

        def zero_body(i, carry):
            for k in range(dh // LANES):
                bufs[0][i, pl.ds(k * LANES, LANES)] = jnp.zeros(
                    (LANES,), jnp.float32)
            return carry

        lax.fori_loop(0, OCH, zero_body, 0)
        for z in range(rows_per_tile // OCH):
            fill_idxz(OCH, r0 + z * OCH)
            pltpu.sync_copy(bufs[0].at[pl.ds(0, OCH)], acc.at[idxz])
        pltpu.make_async_copy(
            h_hbm.at[pl.ds(r0, rows_per_tile), pl.ds(coff, dh)],
            h_sp.at[pl.ds(r0, rows_per_tile)], osem).wait()
        plsc.subcore_barrier()

        base = s * ept

        def col_copy(ch, b):
            return pltpu.make_async_copy(
                col_hbm.at[pl.ds(base + ch * CHUNK, CHUNK)], colv[b], csem[b])

        def row_copy(ch, b):
            return pltpu.make_async_copy(
                row_hbm.at[pl.ds(base + ch * CHUNK, CHUNK)], rowv[b], rsem[b])

        def val_copy(ch, b):
            return pltpu.make_async_copy(
                val_hbm.at[pl.ds(base + ch * CHUNK, CHUNK)], valv[b], vsem[b])

        def gather(b):
            return pltpu.make_async_copy(h_sp.at[colv[b]], bufs[b], gsem[b])

        def idx_issue(ch, b):
            col_copy(ch, b).start()
            val_copy(ch, b).start()

        def scatter_start(b):
            pltpu.async_copy(bufs[b], acc.at[rowv[b]], ssem[b], add=True)

        def scatter_wait(b):
            pltpu.make_async_copy(bufs[b], acc.at[rowv[b]], ssem[b]).wait()

        idx_issue(0, 0)
        idx_issue(1, 1)
        idx_issue(2, 2)
        row_copy(0, 0).start()
        row_copy(1, 1).start()
        col_copy(0, 0).wait()
        gather(0).start()
        col_copy(1, 1).wait()
        gather(1).start()

        def quad_body(i, carry):
            for b in range(NBUF):
                ch = i * NBUF + b
                b2 = (b + 2) % NBUF
                b3 = (b + 3) % NBUF

                @pl.when(ch >= 2)
                def _():
                    scatter_wait(b2)

                @pl.when(ch + 2 < nch)
                def _():
                    row_copy(ch + 2, b2).start()
                    col_copy(ch + 2, b2).wait()
                    gather(b2).start()

                @pl.when(ch + 3 < nch)
                def _():
                    idx_issue(ch + 3, b3)

                gather(b).wait()
                val_copy(ch, b).wait()

                def group_body(g, carry2):
                    vv = valv[b][pl.ds(g * LANES, LANES)]
                    for e in range(LANES):
                        r = g * LANES + e
                        v = vv[e]
                        for k in range(dh // LANES):
                            sl = pl.ds(k * LANES, LANES)
                            bufs[b][r, sl] = bufs[b][r, sl] * v
                    return carry2

                lax.fori_loop(0, CHUNK // LANES, group_body, 0)
                row_copy(ch, b).wait()
                scatter_start(b)
            return carry

        lax.fori_loop(0, nch // NBUF, quad_body, 0)

        scatter_wait((nch - 2) % NBUF)
        scatter_wait((nch - 1) % NBUF)

        plsc.subcore_barrier()

        pltpu.sync_copy(acc.at[pl.ds(r0, rows_per_tile)],
                        out_hbm.at[c, pl.ds(r0, rows_per_tile)])

    return agg(h, col, row, val)


def kernel(features, adj_indices, adj_values, weight, bias, skip_weight):
    n, d_in = features.shape
    d = weight.shape[1]
    e = adj_values.shape[0]

    n_acc = ((n + NS * OCH - 1) // (NS * OCH)) * (NS * OCH)
    feat = jnp.concatenate(
        [features, jnp.zeros((n_acc - n, d_in), jnp.float32)])

    h = pl.pallas_call(
        _matmul_body,
        out_shape=jax.ShapeDtypeStruct((n_acc, d), jnp.float32),
    )(feat, weight)

    group = NS * CHUNK * NBUF
    e_pad = ((e + group - 1) // group) * group
    pad = e_pad - e
    row = adj_indices[0].astype(jnp.int32)
    col = adj_indices[1].astype(jnp.int32)
    val = adj_values
    if pad:
        spread = n + jnp.arange(pad, dtype=jnp.int32) % (n_acc - n)
        row = jnp.concatenate([row, spread])
        col = jnp.concatenate([col, spread])
        val = jnp.concatenate([val, jnp.zeros((pad,), jnp.float32)])

    partials = _sc_aggregate(h, col, row, val, n_acc, d, e_pad)
    partials = partials[:, :n, :]

    blk = 2000
    out = pl.pallas_call(
        _finalize_body,
        grid=(n // blk,),
        in_specs=[
            pl.BlockSpec((blk, d), lambda i: (i, 0)),
            pl.BlockSpec((NC, blk, d // NC), lambda i: (0, i, 0)),
            pl.BlockSpec((1, d), lambda i: (0, 0)),
            pl.BlockSpec((1, d), lambda i: (0, 0)),
        ],
        out_specs=pl.BlockSpec((blk, d), lambda i: (i, 0)),
        out_shape=jax.ShapeDtypeStruct((n, d), jnp.float32),
    )(h[:n], partials, skip_weight.reshape(1, d), bias.reshape(1, d))
    return out


# speedup vs baseline: 10.8773x; 1.0760x over previous
import functools

import jax
import jax.numpy as jnp
from jax import lax
from jax.experimental import pallas as pl
from jax.experimental.pallas import tpu as pltpu
from jax.experimental.pallas import tpu_sc as plsc

NC = 2
NS = 16
LANES = 16
CHUNK = 112
NBUF = 4
OCH = 64
SCH = 32

_SELU_ALPHA = 1.6732632423543772
_SELU_SCALE = 1.0507009873554805


def _matmul_body(x_ref, w_ref, o_ref):
    o_ref[...] = jnp.dot(x_ref[...], w_ref[...],
                         preferred_element_type=jnp.float32)


def _sc_aggregate(h, col, row, val, skip, bias, n_rows, d, e_pad):
    dh = d // NC
    ept = e_pad // NS
    nch = ept // CHUNK
    assert nch % NBUF == 0
    rows_per_tile = n_rows // NS
    assert rows_per_tile % OCH == 0 and rows_per_tile % SCH == 0

    mesh = plsc.VectorSubcoreMesh(core_axis_name="c", subcore_axis_name="s")

    @functools.partial(
        pl.kernel,
        out_type=jax.ShapeDtypeStruct((n_rows, d), jnp.float32),
        mesh=mesh,
        compiler_params=pltpu.CompilerParams(use_tc_tiling_on_sc=False),
        scratch_types=[
            pltpu.VMEM_SHARED((n_rows, dh), jnp.float32),
            pltpu.VMEM_SHARED((n_rows, dh), jnp.float32),
            [pltpu.VMEM((CHUNK,), jnp.int32) for _ in range(NBUF)],
            [pltpu.VMEM((CHUNK,), jnp.int32) for _ in range(NBUF)],
            [pltpu.VMEM((CHUNK,), jnp.float32) for _ in range(NBUF)],
            [pltpu.VMEM((CHUNK, dh), jnp.float32) for _ in range(NBUF)],
            pltpu.VMEM((SCH, d), jnp.float32),
            pltpu.VMEM((OCH,), jnp.int32),
            pltpu.VMEM((dh,), jnp.float32),
            pltpu.VMEM((dh,), jnp.float32),
            pltpu.SemaphoreType.DMA,
            [pltpu.SemaphoreType.DMA for _ in range(NBUF)],
            [pltpu.SemaphoreType.DMA for _ in range(NBUF)],
            [pltpu.SemaphoreType.DMA for _ in range(NBUF)],
            [pltpu.SemaphoreType.DMA for _ in range(NBUF)],
            [pltpu.SemaphoreType.DMA for _ in range(NBUF)],
        ],
    )
    def agg(h_hbm, col_hbm, row_hbm, val_hbm, skip_hbm, bias_hbm, out_hbm,
            h_sp, acc, colv, rowv, valv, bufs, bounce, idxz, skv, biv,
            osem, csem, rsem, vsem, gsem, ssem):
        c = lax.axis_index("c")
        s = lax.axis_index("s")
        r0 = s * rows_per_tile
        coff = c * dh
        iota = lax.iota(jnp.int32, LANES)

        def fill_idxz(nrows, base):
            for g in range(nrows // LANES):
                idxz[pl.ds(g * LANES, LANES)] = iota + (base + g * LANES)

        pltpu.async_copy(
            h_hbm.at[pl.ds(r0, rows_per_tile), pl.ds(coff, dh)],
            h_sp.at[pl.ds(r0, rows_per_tile)], osem)

        def zero_body(i, carry):
            for k in range(dh // LANES):
                bufs[0][i, pl.ds(k * LANES, LANES)] = jnp.zeros(
                    (LANES,), jnp.float32)
            return carry

        lax.fori_loop(0, OCH, zero_body, 0)
        for z in range(rows_per_tile // OCH):
            fill_idxz(OCH, r0 + z * OCH)
            pltpu.sync_copy(bufs[0].at[pl.ds(0, OCH)], acc.at[idxz])
        pltpu.make_async_copy(
            h_hbm.at[pl.ds(r0, rows_per_tile), pl.ds(coff, dh)],
            h_sp.at[pl.ds(r0, rows_per_tile)], osem).wait()
        plsc.subcore_barrier()

        base = s * ept

        def col_copy(ch, b):
            return pltpu.make_async_copy(
                col_hbm.at[pl.ds(base + ch * CHUNK, CHUNK)], colv[b], csem[b])

        def row_copy(ch, b):
            return pltpu.make_async_copy(
                row_hbm.at[pl.ds(base + ch * CHUNK, CHUNK)], rowv[b], rsem[b])

        def val_copy(ch, b):
            return pltpu.make_async_copy(
                val_hbm.at[pl.ds(base + ch * CHUNK, CHUNK)], valv[b], vsem[b])

        def gather(b):
            return pltpu.make_async_copy(h_sp.at[colv[b]], bufs[b], gsem[b])

        def idx_issue(ch, b):
            col_copy(ch, b).start()
            val_copy(ch, b).start()

        def scatter_start(b):
            pltpu.async_copy(bufs[b], acc.at[rowv[b]], ssem[b], add=True)

        def scatter_wait(b):
            pltpu.make_async_copy(bufs[b], acc.at[rowv[b]], ssem[b]).wait()

        idx_issue(0, 0)
        idx_issue(1, 1)
        idx_issue(2, 2)
        row_copy(0, 0).start()
        row_copy(1, 1).start()
        col_copy(0, 0).wait()
        gather(0).start()
        col_copy(1, 1).wait()
        gather(1).start()

        def quad_body(i, carry):
            for b in range(NBUF):
                ch = i * NBUF + b
                b2 = (b + 2) % NBUF
                b3 = (b + 3) % NBUF

                @pl.when(ch >= 2)
                def _():
                    scatter_wait(b2)

                @pl.when(ch + 2 < nch)
                def _():
                    row_copy(ch + 2, b2).start()
                    col_copy(ch + 2, b2).wait()
                    gather(b2).start()

                @pl.when(ch + 3 < nch)
                def _():
                    idx_issue(ch + 3, b3)

                gather(b).wait()
                val_copy(ch, b).wait()

                def group_body(g, carry2):
                    vv = valv[b][pl.ds(g * LANES, LANES)]
                    for e in range(LANES):
                        r = g * LANES + e
                        v = vv[e]
                        for k in range(dh // LANES):
                            sl = pl.ds(k * LANES, LANES)
                            bufs[b][r, sl] = bufs[b][r, sl] * v
                    return carry2

                lax.fori_loop(0, CHUNK // LANES, group_body, 0)
                row_copy(ch, b).wait()
                scatter_start(b)
            return carry

        lax.fori_loop(0, nch // NBUF, quad_body, 0)

        scatter_wait((nch - 2) % NBUF)
        scatter_wait((nch - 1) % NBUF)

        plsc.subcore_barrier()

        pltpu.sync_copy(skip_hbm.at[pl.ds(coff, dh)], skv)
        pltpu.sync_copy(bias_hbm.at[pl.ds(coff, dh)], biv)
        sk = [skv[pl.ds(k * LANES, LANES)] for k in range(dh // LANES)]
        bi = [biv[pl.ds(k * LANES, LANES)] for k in range(dh // LANES)]
        for z in range(rows_per_tile // OCH):
            sl_rows = pl.ds(r0 + z * OCH, OCH)
            pltpu.sync_copy(h_sp.at[sl_rows], bufs[1].at[pl.ds(0, OCH)])
            pltpu.sync_copy(acc.at[sl_rows], bufs[2].at[pl.ds(0, OCH)])

            def fin_body(i, carry):
                for k in range(dh // LANES):
                    sl = pl.ds(k * LANES, LANES)
                    x = bufs[1][i, sl] * sk[k] + bufs[2][i, sl] + bi[k]
                    bufs[3][i, sl] = _SELU_SCALE * jnp.where(
                        x > 0.0, x, _SELU_ALPHA * (jnp.exp(x) - 1.0))
                return carry

            lax.fori_loop(0, OCH, fin_body, 0)
            pltpu.sync_copy(bufs[3].at[pl.ds(0, OCH)],
                            out_hbm.at[sl_rows, pl.ds(coff, dh)])

    return agg(h, col, row, val, skip, bias)


def kernel(features, adj_indices, adj_values, weight, bias, skip_weight):
    n, d_in = features.shape
    d = weight.shape[1]
    e = adj_values.shape[0]

    n_acc = ((n + NS * OCH - 1) // (NS * OCH)) * (NS * OCH)
    feat = jnp.concatenate(
        [features, jnp.zeros((n_acc - n, d_in), jnp.float32)])

    h = pl.pallas_call(
        _matmul_body,
        out_shape=jax.ShapeDtypeStruct((n_acc, d), jnp.float32),
    )(feat, weight)

    group = NS * CHUNK * NBUF
    e_pad = ((e + group - 1) // group) * group
    pad = e_pad - e
    row = adj_indices[0].astype(jnp.int32)
    col = adj_indices[1].astype(jnp.int32)
    val = adj_values
    if pad:
        spread = n + jnp.arange(pad, dtype=jnp.int32) % (n_acc - n)
        row = jnp.concatenate([row, spread])
        col = jnp.concatenate([col, spread])
        val = jnp.concatenate([val, jnp.zeros((pad,), jnp.float32)])

    out = _sc_aggregate(h, col, row, val, skip_weight, bias, n_acc, d, e_pad)
    return out[:n]
